# baseline (speedup 1.0000x reference)
import functools

import numpy as np
import jax
import jax.numpy as jnp
from jax import lax
from jax.experimental import pallas as pl
from jax.experimental.pallas import tpu as pltpu
from jax.experimental.pallas import tpu_sc as plsc

_N = 1024
_NL = 1152
_BT = 128

_NN = [1, 2, 4, 8, 16, 32, 64, 128, 256, 512, 1]
_OFF = [0, 1, 3, 7, 15, 31, 63, 128, 256, 512, 1024]


def _bitrev(k, bits):
    r = 0
    for _ in range(bits):
        r = (r << 1) | (k & 1)
        k >>= 1
    return r


_node_of_pos = np.full((_NL,), -1, dtype=np.int64)
_pos_of_node = np.zeros((_N,), dtype=np.int64)
for _d in range(11):
    _bits = _d if _d < 10 else 0
    for _k in range(_NN[_d]):
        _node = 2 ** _d - 1 + _k
        _pos = _OFF[_d] + _bitrev(_k, _bits)
        _node_of_pos[_pos] = _node
        _pos_of_node[_node] = _pos

_parent_np = np.zeros((_N,), dtype=np.int64)
for _i in range(1, _N):
    _parent_np[_i] = (_i - 1) // 2

_perm_in = np.zeros((2 * _NL,), dtype=np.int32)
for _q in range(_NL):
    _nd = _node_of_pos[_q]
    _perm_in[_q] = 2 * _nd if _nd >= 0 else 0
    _perm_in[_NL + _q] = 2 * _nd + 1 if _nd >= 0 else 0
_perm_out = np.zeros((2 * _N,), dtype=np.int32)
for _nd in range(_N):
    _perm_out[2 * _nd] = _pos_of_node[_nd]
    _perm_out[2 * _nd + 1] = _NL + _pos_of_node[_nd]

_ROWIDX = np.zeros((2 * _NL,), dtype=np.int32)
_LANE0 = np.zeros((2, _NL), dtype=np.int32)
for _s in range(2 * _NL):
    _dir, _q = _s // _NL, _s % _NL
    _j = _node_of_pos[_q]
    if _j >= 1:
        if _dir == 0:
            _i, _m = _parent_np[_j], _j
        else:
            _i, _m = _j, _parent_np[_j]
        _ROWIDX[_s] = _i * 32 + _m // 32
        _LANE0[_dir, _q] = (_m % 32) * 4


def _sc_gather_rows(table, rowidx):
    mesh = plsc.VectorSubcoreMesh(core_axis_name="c", subcore_axis_name="s")

    @functools.partial(
        pl.kernel,
        mesh=mesh,
        out_type=jax.ShapeDtypeStruct((2 * _NL, 128), jnp.float32),
        scratch_types=[
            pltpu.VMEM((72,), jnp.int32),
            pltpu.VMEM((72, 128), jnp.float32),
            pltpu.SemaphoreType.DMA,
        ],
    )
    def k(table_hbm, rowidx_hbm, out_hbm, ridx_v, rows_v, sem):
        wid = lax.axis_index("s") * 2 + lax.axis_index("c")
        base = wid * 72
        pltpu.sync_copy(rowidx_hbm.at[pl.ds(base, 72)], ridx_v)
        cp = pltpu.async_copy(table_hbm.at[ridx_v], rows_v, sem)
        cp.wait()
        pltpu.sync_copy(rows_v, out_hbm.at[pl.ds(base, 72)])

    return k(table, rowidx)


def _extract_body(r_ref, lane_ref, o_ref):
    R = r_ref[:, :].reshape(2, _NL, 128)
    lane = lane_ref[:, :].reshape(2, _NL, 1)
    li = lax.broadcasted_iota(jnp.int32, (2, _NL, 128), 2)
    rows = []
    for dirab in range(8):
        d, ab = dirab // 4, dirab % 4
        m = (li[d] == lane[d] + ab).astype(jnp.float32)
        rows.append(jnp.sum(R[d] * m, axis=-1)[None, :])
    o_ref[:, :] = jnp.concatenate(rows, axis=0)


def _tc_extract(R):
    return pl.pallas_call(
        _extract_body,
        out_shape=jax.ShapeDtypeStruct((8, _NL), jnp.float32),
    )(R, jnp.asarray(_LANE0))


def _lse2(u, v):
    m = jnp.maximum(u, v)
    return m + jnp.log(jnp.exp(u - m) + jnp.exp(v - m))


def _tc_body(x_ref, p_ref, o_ref):
    X = x_ref[:, :]
    P = p_ref[:, :]
    bt = X.shape[0]
    X0 = X[:, :_NL]
    X1 = X[:, _NL:]

    def pr(row, d):
        return P[row:row + 1, _OFF[d]:_OFF[d] + _NN[d]]

    def xs(Xl, d):
        return Xl[:, _OFF[d]:_OFF[d] + _NN[d]]

    a0 = {}
    a1 = {}
    m0 = _lse2(xs(X0, 10) + pr(0, 10), xs(X1, 10) + pr(1, 10))
    m1 = _lse2(xs(X0, 10) + pr(2, 10), xs(X1, 10) + pr(3, 10))
    z511 = jnp.zeros((bt, 511), jnp.float32)
    a0[9] = jnp.concatenate([m0, z511], axis=1)
    a1[9] = jnp.concatenate([m1, z511], axis=1)
    for d in range(9, 0, -1):
        n = _NN[d]
        l0 = xs(X0, d)
        l1 = xs(X1, d)
        if d in a0:
            l0 = l0 + a0[d]
            l1 = l1 + a1[d]
        m0 = _lse2(l0 + pr(0, d), l1 + pr(1, d))
        m1 = _lse2(l0 + pr(2, d), l1 + pr(3, d))
        h = n // 2
        a0[d - 1] = m0[:, :h] + m0[:, h:]
        a1[d - 1] = m1[:, :h] + m1[:, h:]

    b0 = {0: jnp.zeros((bt, 1), jnp.float32)}
    b1 = {0: jnp.zeros((bt, 1), jnp.float32)}
    for d in range(0, 9):
        pl0 = xs(X0, d) + b0[d]
        pl1 = xs(X1, d) + b1[d]
        r0 = jnp.concatenate([pl0, pl0], axis=1)
        r1 = jnp.concatenate([pl1, pl1], axis=1)
        b0[d + 1] = _lse2(r0 + pr(4, d + 1), r1 + pr(5, d + 1))
        b1[d + 1] = _lse2(r0 + pr(6, d + 1), r1 + pr(7, d + 1))
    pl0 = X0[:, 512:513] + b0[9][:, 0:1]
    pl1 = X1[:, 512:513] + b1[9][:, 0:1]
    b0[10] = _lse2(pl0 + pr(4, 10), pl1 + pr(5, 10))
    b1[10] = _lse2(pl0 + pr(6, 10), pl1 + pr(7, 10))

    a0[10] = jnp.zeros((bt, 1), jnp.float32)
    a1[10] = jnp.zeros((bt, 1), jnp.float32)
    zpad1 = jnp.zeros((bt, 1), jnp.float32)
    zpad127 = jnp.zeros((bt, 127), jnp.float32)

    def asm(parts):
        seq = [parts[d] for d in range(7)] + [zpad1]
        seq += [parts[d] for d in range(7, 11)] + [zpad127]
        return jnp.concatenate(seq, axis=1)

    S0 = X0 + asm(a0) + asm(b0)
    S1 = X1 + asm(a1) + asm(b1)
    z = _lse2(S0, S1)
    o_ref[:, :] = jnp.concatenate([S0 - z, S1 - z], axis=1)


def _tc_bp(Xp, ptab):
    B = Xp.shape[0]
    return pl.pallas_call(
        _tc_body,
        grid=(B // _BT,),
        in_specs=[
            pl.BlockSpec((_BT, 2 * _NL), lambda i: (i, 0)),
            pl.BlockSpec((8, _NL), lambda i: (0, 0)),
        ],
        out_specs=pl.BlockSpec((_BT, 2 * _NL), lambda i: (i, 0)),
        out_shape=jax.ShapeDtypeStruct((B, 2 * _NL), jnp.float32),
    )(Xp, ptab)


def kernel(X, pairs):
    B = X.shape[0]
    Xp = jnp.take(X.reshape(B, 2 * _N).T, jnp.asarray(_perm_in), axis=0).T
    R = _sc_gather_rows(pairs.reshape(32768, 128), jnp.asarray(_ROWIDX))
    out_p = _tc_bp(Xp, _tc_extract(R))
    out = jnp.take(out_p.T, jnp.asarray(_perm_out), axis=0).T
    return out.reshape(B, _N, 2)


# speedup vs baseline: 23.0993x; 23.0993x over previous
import functools

import numpy as np
import jax
import jax.numpy as jnp
from jax import lax
from jax.experimental import pallas as pl
from jax.experimental.pallas import tpu as pltpu
from jax.experimental.pallas import tpu_sc as plsc

_N = 1024
_NL = 1152
_BT = 128

_NN = [1, 2, 4, 8, 16, 32, 64, 128, 256, 512, 1]
_OFF = [0, 1, 3, 7, 15, 31, 63, 128, 256, 512, 1024]


def _bitrev(k, bits):
    r = 0
    for _ in range(bits):
        r = (r << 1) | (k & 1)
        k >>= 1
    return r


_node_of_pos = np.full((_NL,), -1, dtype=np.int64)
_pos_of_node = np.zeros((_N,), dtype=np.int64)
for _d in range(11):
    _bits = _d if _d < 10 else 0
    for _k in range(_NN[_d]):
        _node = 2 ** _d - 1 + _k
        _pos = _OFF[_d] + _bitrev(_k, _bits)
        _node_of_pos[_pos] = _node
        _pos_of_node[_node] = _pos

_parent_np = np.zeros((_N,), dtype=np.int64)
for _i in range(1, _N):
    _parent_np[_i] = (_i - 1) // 2

_perm_in = np.zeros((2 * _NL,), dtype=np.int32)
for _q in range(_NL):
    _nd = _node_of_pos[_q]
    _perm_in[_q] = 2 * _nd if _nd >= 0 else 0
    _perm_in[_NL + _q] = 2 * _nd + 1 if _nd >= 0 else 0
_perm_out = np.zeros((2 * _N,), dtype=np.int32)
for _nd in range(_N):
    _perm_out[2 * _nd] = _pos_of_node[_nd]
    _perm_out[2 * _nd + 1] = _NL + _pos_of_node[_nd]

_ROWIDX = np.zeros((8 * _NL,), dtype=np.int32)
_LANE0 = np.zeros((8, _NL), dtype=np.int32)
for _e in range(8 * _NL):
    _dirab, _q = _e // _NL, _e % _NL
    _j = _node_of_pos[_q]
    if _j >= 1:
        _a, _b = (_dirab % 4) // 2, (_dirab % 4) % 2
        if _dirab < 4:
            _i, _m = _parent_np[_j], _j
        else:
            _i, _m = _j, _parent_np[_j]
        _ROWIDX[_e] = ((_i * 2 + _a) * 8 + _m // 128) * 2 + _b
        _LANE0[_dirab, _q] = _m % 128


def _pairs_flat_view(pairs):
    v = pairs.transpose(0, 2, 3, 1)
    v = v.reshape(_N, 2, 2, 8, 128)
    v = v.transpose(0, 1, 3, 2, 4)
    return v.reshape(32768, 128)


def _sc_gather_rows(table, rowidx):
    mesh = plsc.VectorSubcoreMesh(core_axis_name="c", subcore_axis_name="s")

    @functools.partial(
        pl.kernel,
        mesh=mesh,
        out_type=jax.ShapeDtypeStruct((8 * _NL, 128), jnp.float32),
        scratch_types=[
            pltpu.VMEM((288,), jnp.int32),
            pltpu.VMEM((288, 128), jnp.float32),
            pltpu.SemaphoreType.DMA,
        ],
    )
    def k(table_hbm, rowidx_hbm, out_hbm, ridx_v, rows_v, sem):
        wid = lax.axis_index("s") * 2 + lax.axis_index("c")
        base = wid * 288
        pltpu.sync_copy(rowidx_hbm.at[pl.ds(base, 288)], ridx_v)
        copies = [
            pltpu.async_copy(
                table_hbm.at[ridx_v.at[pl.ds(c * 96, 96)]],
                rows_v.at[pl.ds(c * 96, 96)],
                sem,
            )
            for c in range(3)
        ]
        for c in copies:
            c.wait()
        pltpu.sync_copy(rows_v, out_hbm.at[pl.ds(base, 288)])

    return k(table, rowidx)


def _extract_body(r_ref, lane_ref, o_ref):
    R = r_ref[:, :].reshape(8, _NL, 128)
    lane = lane_ref[:, :].reshape(8, _NL, 1)
    li = lax.broadcasted_iota(jnp.int32, (8, _NL, 128), 2)
    m = (li == lane).astype(jnp.float32)
    o_ref[:, :] = jnp.sum(R * m, axis=-1)


def _tc_extract(R):
    return pl.pallas_call(
        _extract_body,
        out_shape=jax.ShapeDtypeStruct((8, _NL), jnp.float32),
    )(R, jnp.asarray(_LANE0))


def _lse2(u, v):
    m = jnp.maximum(u, v)
    return m + jnp.log(jnp.exp(u - m) + jnp.exp(v - m))


def _tc_body(x_ref, p_ref, o_ref):
    X = x_ref[:, :]
    P = p_ref[:, :]
    bt = X.shape[0]
    X0 = X[:, :_NL]
    X1 = X[:, _NL:]

    def pr(row, d):
        return P[row:row + 1, _OFF[d]:_OFF[d] + _NN[d]]

    def xs(Xl, d):
        return Xl[:, _OFF[d]:_OFF[d] + _NN[d]]

    a0 = {}
    a1 = {}
    m0 = _lse2(xs(X0, 10) + pr(0, 10), xs(X1, 10) + pr(1, 10))
    m1 = _lse2(xs(X0, 10) + pr(2, 10), xs(X1, 10) + pr(3, 10))
    z511 = jnp.zeros((bt, 511), jnp.float32)
    a0[9] = jnp.concatenate([m0, z511], axis=1)
    a1[9] = jnp.concatenate([m1, z511], axis=1)
    for d in range(9, 0, -1):
        n = _NN[d]
        l0 = xs(X0, d)
        l1 = xs(X1, d)
        if d in a0:
            l0 = l0 + a0[d]
            l1 = l1 + a1[d]
        m0 = _lse2(l0 + pr(0, d), l1 + pr(1, d))
        m1 = _lse2(l0 + pr(2, d), l1 + pr(3, d))
        h = n // 2
        a0[d - 1] = m0[:, :h] + m0[:, h:]
        a1[d - 1] = m1[:, :h] + m1[:, h:]

    b0 = {0: jnp.zeros((bt, 1), jnp.float32)}
    b1 = {0: jnp.zeros((bt, 1), jnp.float32)}
    for d in range(0, 9):
        pl0 = xs(X0, d) + b0[d]
        pl1 = xs(X1, d) + b1[d]
        r0 = jnp.concatenate([pl0, pl0], axis=1)
        r1 = jnp.concatenate([pl1, pl1], axis=1)
        b0[d + 1] = _lse2(r0 + pr(4, d + 1), r1 + pr(5, d + 1))
        b1[d + 1] = _lse2(r0 + pr(6, d + 1), r1 + pr(7, d + 1))
    pl0 = X0[:, 512:513] + b0[9][:, 0:1]
    pl1 = X1[:, 512:513] + b1[9][:, 0:1]
    b0[10] = _lse2(pl0 + pr(4, 10), pl1 + pr(5, 10))
    b1[10] = _lse2(pl0 + pr(6, 10), pl1 + pr(7, 10))

    a0[10] = jnp.zeros((bt, 1), jnp.float32)
    a1[10] = jnp.zeros((bt, 1), jnp.float32)
    zpad1 = jnp.zeros((bt, 1), jnp.float32)
    zpad127 = jnp.zeros((bt, 127), jnp.float32)

    def asm(parts):
        seq = [parts[d] for d in range(7)] + [zpad1]
        seq += [parts[d] for d in range(7, 11)] + [zpad127]
        return jnp.concatenate(seq, axis=1)

    S0 = X0 + asm(a0) + asm(b0)
    S1 = X1 + asm(a1) + asm(b1)
    z = _lse2(S0, S1)
    o_ref[:, :] = jnp.concatenate([S0 - z, S1 - z], axis=1)


def _tc_bp(Xp, ptab):
    B = Xp.shape[0]
    return pl.pallas_call(
        _tc_body,
        grid=(B // _BT,),
        in_specs=[
            pl.BlockSpec((_BT, 2 * _NL), lambda i: (i, 0)),
            pl.BlockSpec((8, _NL), lambda i: (0, 0)),
        ],
        out_specs=pl.BlockSpec((_BT, 2 * _NL), lambda i: (i, 0)),
        out_shape=jax.ShapeDtypeStruct((B, 2 * _NL), jnp.float32),
    )(Xp, ptab)


def kernel(X, pairs):
    B = X.shape[0]
    Xp = jnp.take(X.reshape(B, 2 * _N).T, jnp.asarray(_perm_in), axis=0).T
    R = _sc_gather_rows(_pairs_flat_view(pairs), jnp.asarray(_ROWIDX))
    out_p = _tc_bp(Xp, _tc_extract(R))
    out = jnp.take(out_p.T, jnp.asarray(_perm_out), axis=0).T
    return out.reshape(B, _N, 2)
